# Initial kernel scaffold; baseline (speedup 1.0000x reference)
import functools

import jax
import jax.numpy as jnp
from jax.experimental import pallas as pl

N = 50000
E = 800000
ET = 100000
DIN = 74
HID = 64
H = 4
NEA = 5
HL = 128
OUT = 4


def _gcn5(h, src, dst, edge_attr, W, b, n):
    outs = []
    loop = jnp.arange(n, dtype=src.dtype)
    s = jnp.concatenate([src, loop])
    d = jnp.concatenate([dst, loop])
    for i in range(NEA):
        ew = edge_attr[:, i]
        w = jnp.concatenate([ew, jnp.ones((n,), h.dtype)])
        deg = jax.ops.segment_sum(w, d, num_segments=n)
        dinv = jnp.where(deg > 0, 1.0 / jnp.sqrt(jnp.maximum(deg, 1e-12)), 0.0)
        norm = dinv[s] * w * dinv[d]
        hw = h @ W[i]
        out = jax.ops.segment_sum(hw[s] * norm[:, None], d, num_segments=n)
        outs.append(out + b[i])
    return jnp.concatenate(outs, axis=-1)


def _gat(x, src, dst, Wg, att_src, att_dst, bias, n):
    h = (x @ Wg).reshape(n, H, HID)
    a_s = jnp.sum(h * att_src, axis=-1)
    a_d = jnp.sum(h * att_dst, axis=-1)
    loop = jnp.arange(n, dtype=src.dtype)
    s = jnp.concatenate([src, loop])
    d = jnp.concatenate([dst, loop])
    alpha = jax.nn.leaky_relu(a_s[s] + a_d[d], 0.2)
    amax = jax.ops.segment_max(alpha, d, num_segments=n)
    amax = jnp.where(jnp.isfinite(amax), amax, 0.0)
    ex = jnp.exp(alpha - amax[d])
    den = jax.ops.segment_sum(ex, d, num_segments=n)
    coef = ex / (den[d] + 1e-16)
    out = jax.ops.segment_sum(h[s] * coef[:, :, None], d, num_segments=n)
    return out.reshape(n, H * HID) + bias


def _mlp_body(e1_ref, e2_ref, w0_ref, b0_ref, w1_ref, b1_ref, w2_ref, b2_ref,
              w3_ref, b3_ref, w4_ref, b4_ref, o_ref):
    def mlp(e):
        z = jnp.maximum(e @ w0_ref[...] + b0_ref[...], 0.0)
        z = jnp.maximum(z @ w1_ref[...] + b1_ref[...], 0.0)
        z = z @ w2_ref[...] + b2_ref[...]
        z = jnp.maximum(z @ w3_ref[...] + b3_ref[...], 0.0)
        return z @ w4_ref[...] + b4_ref[...]

    t1 = mlp(e1_ref[...])
    t2 = mlp(e2_ref[...])
    o_ref[...] = 0.5 * (t1 + t2[:, jnp.array([0, 2, 1, 3])])


def _link_mlp(e1, e2, W_l0, b_l0, W_l1, b_l1, W_l2, b_l2, W_l3, b_l3, W_l4, b_l4):
    blk = 2000
    grid = (ET // blk,)
    spec_e = pl.BlockSpec((blk, 2 * NEA * HID), lambda i: (i, 0))
    full = lambda *s: pl.BlockSpec(s, lambda i: (0,) * len(s))
    return pl.pallas_call(
        _mlp_body,
        grid=grid,
        in_specs=[spec_e, spec_e,
                  full(2 * NEA * HID, HL), full(HL), full(HL, HL), full(HL),
                  full(HL, HL), full(HL), full(HL, HL), full(HL),
                  full(HL, OUT), full(OUT)],
        out_specs=pl.BlockSpec((blk, OUT), lambda i: (i, 0)),
        out_shape=jax.ShapeDtypeStruct((ET, OUT), jnp.float32),
    )(e1, e2, W_l0, b_l0, W_l1, b_l1, W_l2, b_l2, W_l3, b_l3, W_l4, b_l4)


def kernel(x, edge_index, edge_attr, edge_index_test, Wg, att_src, att_dst, bg,
           W_gcnA, b_gcnA, W_gcnB, b_gcnB,
           W_l0, b_l0, W_l1, b_l1, W_l2, b_l2, W_l3, b_l3, W_l4, b_l4):
    src, dst = edge_index[0], edge_index[1]
    h = jax.nn.relu(_gat(x, src, dst, Wg, att_src, att_dst, bg, N))
    h = jax.nn.relu(_gcn5(h, src, dst, edge_attr, W_gcnA, b_gcnA, N))
    for l in range(4):
        h = jax.nn.relu(_gcn5(h, src, dst, edge_attr, W_gcnB[l], b_gcnB[l], N))
    s2, d2 = edge_index_test[0], edge_index_test[1]
    e1 = jnp.concatenate([h[s2], h[d2]], axis=-1)
    e2 = jnp.concatenate([h[d2], h[s2]], axis=-1)
    return _link_mlp(e1, e2, W_l0, b_l0, W_l1, b_l1, W_l2, b_l2, W_l3, b_l3,
                     W_l4, b_l4)


# speedup vs baseline: 1.0018x; 1.0018x over previous
import functools

import jax
import jax.numpy as jnp
from jax.experimental import pallas as pl

N = 50000
E = 800000
ET = 100000
DIN = 74
HID = 64
H = 4
NEA = 5
HL = 128
OUT = 4


def _gcn5(h, src, dst, edge_attr, W, b, n):
    outs = []
    loop = jnp.arange(n, dtype=src.dtype)
    s = jnp.concatenate([src, loop])
    d = jnp.concatenate([dst, loop])
    for i in range(NEA):
        ew = edge_attr[:, i]
        w = jnp.concatenate([ew, jnp.ones((n,), h.dtype)])
        deg = jax.ops.segment_sum(w, d, num_segments=n)
        dinv = jnp.where(deg > 0, 1.0 / jnp.sqrt(jnp.maximum(deg, 1e-12)), 0.0)
        norm = dinv[s] * w * dinv[d]
        hw = h @ W[i]
        out = jax.ops.segment_sum(hw[s] * norm[:, None], d, num_segments=n)
        outs.append(out + b[i])
    return jnp.concatenate(outs, axis=-1)


def _gat(x, src, dst, Wg, att_src, att_dst, bias, n):
    h = (x @ Wg).reshape(n, H, HID)
    a_s = jnp.sum(h * att_src, axis=-1)
    a_d = jnp.sum(h * att_dst, axis=-1)
    loop = jnp.arange(n, dtype=src.dtype)
    s = jnp.concatenate([src, loop])
    d = jnp.concatenate([dst, loop])
    alpha = jax.nn.leaky_relu(a_s[s] + a_d[d], 0.2)
    amax = jax.ops.segment_max(alpha, d, num_segments=n)
    amax = jnp.where(jnp.isfinite(amax), amax, 0.0)
    ex = jnp.exp(alpha - amax[d])
    den = jax.ops.segment_sum(ex, d, num_segments=n)
    coef = ex / (den[d] + 1e-16)
    out = jax.ops.segment_sum(h[s] * coef[:, :, None], d, num_segments=n)
    return out.reshape(n, H * HID) + bias


def _mlp_body(e1_ref, e2_ref, w0_ref, b0_ref, w1_ref, b1_ref, w2_ref, b2_ref,
              w3_ref, b3_ref, w4_ref, b4_ref, o_ref):
    def mlp(e):
        z = jnp.maximum(e @ w0_ref[...] + b0_ref[...], 0.0)
        z = jnp.maximum(z @ w1_ref[...] + b1_ref[...], 0.0)
        z = z @ w2_ref[...] + b2_ref[...]
        z = jnp.maximum(z @ w3_ref[...] + b3_ref[...], 0.0)
        return z @ w4_ref[...] + b4_ref[...]

    t1 = mlp(e1_ref[...])
    t2 = mlp(e2_ref[...])
    t2p = jnp.concatenate(
        [t2[:, 0:1], t2[:, 2:3], t2[:, 1:2], t2[:, 3:4]], axis=1)
    o_ref[...] = 0.5 * (t1 + t2p)


def _link_mlp(e1, e2, W_l0, b_l0, W_l1, b_l1, W_l2, b_l2, W_l3, b_l3, W_l4, b_l4):
    blk = 2000
    grid = (ET // blk,)
    spec_e = pl.BlockSpec((blk, 2 * NEA * HID), lambda i: (i, 0))
    full = lambda *s: pl.BlockSpec(s, lambda i: (0,) * len(s))
    return pl.pallas_call(
        _mlp_body,
        grid=grid,
        in_specs=[spec_e, spec_e,
                  full(2 * NEA * HID, HL), full(HL), full(HL, HL), full(HL),
                  full(HL, HL), full(HL), full(HL, HL), full(HL),
                  full(HL, OUT), full(OUT)],
        out_specs=pl.BlockSpec((blk, OUT), lambda i: (i, 0)),
        out_shape=jax.ShapeDtypeStruct((ET, OUT), jnp.float32),
    )(e1, e2, W_l0, b_l0, W_l1, b_l1, W_l2, b_l2, W_l3, b_l3, W_l4, b_l4)


def kernel(x, edge_index, edge_attr, edge_index_test, Wg, att_src, att_dst, bg,
           W_gcnA, b_gcnA, W_gcnB, b_gcnB,
           W_l0, b_l0, W_l1, b_l1, W_l2, b_l2, W_l3, b_l3, W_l4, b_l4):
    src, dst = edge_index[0], edge_index[1]
    h = jax.nn.relu(_gat(x, src, dst, Wg, att_src, att_dst, bg, N))
    h = jax.nn.relu(_gcn5(h, src, dst, edge_attr, W_gcnA, b_gcnA, N))
    for l in range(4):
        h = jax.nn.relu(_gcn5(h, src, dst, edge_attr, W_gcnB[l], b_gcnB[l], N))
    s2, d2 = edge_index_test[0], edge_index_test[1]
    e1 = jnp.concatenate([h[s2], h[d2]], axis=-1)
    e2 = jnp.concatenate([h[d2], h[s2]], axis=-1)
    return _link_mlp(e1, e2, W_l0, b_l0, W_l1, b_l1, W_l2, b_l2, W_l3, b_l3,
                     W_l4, b_l4)


# speedup vs baseline: 9.0533x; 9.0367x over previous
import functools

import jax
import jax.numpy as jnp
from jax import lax
from jax.experimental import pallas as pl
from jax.experimental.pallas import tpu as pltpu
from jax.experimental.pallas import tpu_sc as plsc

N = 50000
E = 800000
ET = 100000
DIN = 74
HID = 64
H = 4
NEA = 5
HL = 128
OUT = 4

BLK = 1000
NBLK = N // BLK
B = 128
E2 = 811008
NBAT = E2 // B
ETP = 102400
NPAD = 16
NR = 50048
NCHUNK = NR // 128
KCH = 25


def _m1_body(xp_ref, wg_ref, atts_ref, attd_ref, hp_ref, ad_ref):
    h = jnp.dot(xp_ref[...], wg_ref[...], preferred_element_type=jnp.float32)
    ps = h * atts_ref[...]
    pd = h * attd_ref[...]
    cols = []
    for hd in range(H):
        cols.append(jnp.sum(ps[:, 64 * hd:64 * hd + 64], axis=1, keepdims=True))
    for hd in range(H):
        cols.append(jnp.sum(pd[:, 64 * hd:64 * hd + 64], axis=1, keepdims=True))
    a = jnp.concatenate(cols, axis=1)
    pad = jnp.zeros_like(a)
    ad_ref[...] = jnp.concatenate([a, pad], axis=1)
    for p in range(8):
        hp_ref[p] = h[:, 32 * p:32 * p + 32]


def _m1(xp, wgp, atts, attd):
    return pl.pallas_call(
        _m1_body,
        grid=(NBLK,),
        in_specs=[pl.BlockSpec((BLK, 128), lambda i: (i, 0)),
                  pl.BlockSpec((128, 256), lambda i: (0, 0)),
                  pl.BlockSpec((1, 256), lambda i: (0, 0)),
                  pl.BlockSpec((1, 256), lambda i: (0, 0))],
        out_specs=[pl.BlockSpec((8, BLK, 32), lambda i: (0, i, 0)),
                   pl.BlockSpec((BLK, 16), lambda i: (i, 0))],
        out_shape=[jax.ShapeDtypeStruct((8, N, 32), jnp.float32),
                   jax.ShapeDtypeStruct((N, 16), jnp.float32)],
    )(xp, wgp, atts, attd)


def _gatepi_body(num_ref, hp_ref, ad_ref, den_ref, bg_ref, wcat_ref,
                 hpa_ref, dv_ref):
    ad = ad_ref[...]
    dent = den_ref[0] + den_ref[1]
    a_sum = ad[:, 0:4] + ad[:, 4:8]
    lk = jnp.maximum(a_sum, 0.0) + 0.2 * jnp.minimum(a_sum, 0.0)
    exself = jnp.exp(lk)
    den = dent[:, 0:4] + exself + 1e-16
    deg = dent[:, 4:9] + 1.0
    dv = lax.rsqrt(deg)
    segs = []
    for p in range(8):
        hd = p // 2
        t = (num_ref[p] + exself[:, hd:hd + 1] * hp_ref[p]) / den[:, hd:hd + 1]
        segs.append(t)
    h0 = jnp.concatenate(segs, axis=1) + bg_ref[...]
    h0 = jnp.maximum(h0, 0.0)
    hpa = jnp.dot(h0, wcat_ref[...], preferred_element_type=jnp.float32)
    for p in range(10):
        i = p // 2
        hpa_ref[p] = hpa[:, 32 * p:32 * p + 32] * dv[:, i:i + 1]
    dv_ref[...] = jnp.concatenate([dv, jnp.zeros_like(dv[:, 0:3])], axis=1)


def _gatepi(num_parts, hp_parts, ad, den2, bg2, wcat_a):
    return pl.pallas_call(
        _gatepi_body,
        grid=(NBLK,),
        in_specs=[pl.BlockSpec((8, BLK, 32), lambda i: (0, i, 0)),
                  pl.BlockSpec((8, BLK, 32), lambda i: (0, i, 0)),
                  pl.BlockSpec((BLK, 16), lambda i: (i, 0)),
                  pl.BlockSpec((2, BLK, 16), lambda i: (0, i, 0)),
                  pl.BlockSpec((1, 256), lambda i: (0, 0)),
                  pl.BlockSpec((256, 320), lambda i: (0, 0))],
        out_specs=[pl.BlockSpec((10, BLK, 32), lambda i: (0, i, 0)),
                   pl.BlockSpec((BLK, 8), lambda i: (i, 0))],
        out_shape=[jax.ShapeDtypeStruct((10, N, 32), jnp.float32),
                   jax.ShapeDtypeStruct((N, 8), jnp.float32)],
    )(num_parts, hp_parts, ad, den2, bg2, wcat_a)


def _layer_body(s_ref, hp_ref, dv_ref, b_ref, wcat_ref, out_ref):
    dv = dv_ref[...]
    segs = []
    for p in range(10):
        i = p // 2
        t = dv[:, i:i + 1] * (s_ref[p] + hp_ref[p]) + b_ref[0, 32 * p:32 * p + 32]
        segs.append(t)
    h = jnp.maximum(jnp.concatenate(segs, axis=1), 0.0)
    hn = jnp.dot(h, wcat_ref[...], preferred_element_type=jnp.float32)
    for p in range(10):
        i = p // 2
        out_ref[p] = hn[:, 32 * p:32 * p + 32] * dv[:, i:i + 1]


def _layer(s_parts, hp_parts, dv, b2, wcat):
    return pl.pallas_call(
        _layer_body,
        grid=(NBLK,),
        in_specs=[pl.BlockSpec((10, BLK, 32), lambda i: (0, i, 0)),
                  pl.BlockSpec((10, BLK, 32), lambda i: (0, i, 0)),
                  pl.BlockSpec((BLK, 8), lambda i: (i, 0)),
                  pl.BlockSpec((1, 320), lambda i: (0, 0)),
                  pl.BlockSpec((320, 320), lambda i: (0, 0))],
        out_specs=pl.BlockSpec((10, BLK, 32), lambda i: (0, i, 0)),
        out_shape=jax.ShapeDtypeStruct((10, N, 32), jnp.float32),
    )(s_parts, hp_parts, dv, b2, wcat)


def _final_body(s_ref, hp_ref, dv_ref, b_ref, out_ref):
    dv = dv_ref[...]
    segs = []
    for p in range(10):
        i = p // 2
        t = dv[:, i:i + 1] * (s_ref[p] + hp_ref[p]) + b_ref[0, 32 * p:32 * p + 32]
        segs.append(t)
    out_ref[...] = jnp.maximum(jnp.concatenate(segs, axis=1), 0.0)


def _final(s_parts, hp_parts, dv, b2):
    return pl.pallas_call(
        _final_body,
        grid=(NBLK,),
        in_specs=[pl.BlockSpec((10, BLK, 32), lambda i: (0, i, 0)),
                  pl.BlockSpec((10, BLK, 32), lambda i: (0, i, 0)),
                  pl.BlockSpec((BLK, 8), lambda i: (i, 0)),
                  pl.BlockSpec((1, 320), lambda i: (0, 0))],
        out_specs=pl.BlockSpec((BLK, 320), lambda i: (i, 0)),
        out_shape=jax.ShapeDtypeStruct((N, 320), jnp.float32),
    )(s_parts, hp_parts, dv, b2)


def _mlp_body(g1_ref, g2_ref, w0_ref, b0_ref, w1_ref, b1_ref, w2_ref, b2_ref,
              w3_ref, b3_ref, w4_ref, b4_ref, o_ref):
    def mlp(e):
        z = jnp.maximum(
            jnp.dot(e, w0_ref[...], preferred_element_type=jnp.float32)
            + b0_ref[...], 0.0)
        z = jnp.maximum(
            jnp.dot(z, w1_ref[...], preferred_element_type=jnp.float32)
            + b1_ref[...], 0.0)
        z = jnp.dot(z, w2_ref[...], preferred_element_type=jnp.float32) + b2_ref[...]
        z = jnp.maximum(
            jnp.dot(z, w3_ref[...], preferred_element_type=jnp.float32)
            + b3_ref[...], 0.0)
        return jnp.dot(z, w4_ref[...], preferred_element_type=jnp.float32) + b4_ref[...]

    g1 = g1_ref[...]
    g2 = g2_ref[...]
    t1 = mlp(jnp.concatenate([g1, g2], axis=1))
    t2 = mlp(jnp.concatenate([g2, g1], axis=1))
    t2p = jnp.concatenate(
        [t2[:, 0:1], t2[:, 2:3], t2[:, 1:2], t2[:, 3:4]], axis=1)
    o_ref[...] = 0.5 * (t1 + t2p)


def _mlp(g1, g2, w0, b0, w1, b1, w2, b2, w3, b3, w4, b4):
    mblk = 2048
    spec_g = pl.BlockSpec((mblk, 320), lambda i: (i, 0))
    full = lambda *s: pl.BlockSpec(s, lambda i: (0,) * len(s))
    return pl.pallas_call(
        _mlp_body,
        grid=(ETP // mblk,),
        in_specs=[spec_g, spec_g,
                  full(640, HL), full(1, HL), full(HL, HL), full(1, HL),
                  full(HL, HL), full(1, HL), full(HL, HL), full(1, HL),
                  full(HL, OUT), full(1, OUT)],
        out_specs=pl.BlockSpec((mblk, OUT), lambda i: (i, 0)),
        out_shape=jax.ShapeDtypeStruct((ETP, OUT), jnp.float32),
    )(g1, g2, w0, b0, w1, b1, w2, b2, w3, b3, w4, b4)


_MESH = plsc.VectorSubcoreMesh(core_axis_name="c", subcore_axis_name="s")
_LANE = lambda: lax.broadcasted_iota(jnp.int32, (16,), 0)


def _zero_vmem(ref, rows, cols):
    z = jnp.zeros((16,), jnp.float32)
    for r in range(rows):
        for j in range(cols // 16):
            ref[r, pl.ds(16 * j, 16)] = z


def _s1_kernel(srch, dsth, eah, adh, coefh, denh,
               sbuf0, dbuf0, eabuf0, adsrc0, addst0, comb0,
               sbuf1, dbuf1, eabuf1, adsrc1, addst1, comb1,
               zbuf, bounce, acc, lsem0, lsem1, gsem):
    cid = lax.axis_index("c")
    sid = lax.axis_index("s")
    wid = sid * 2 + cid

    _zero_vmem(zbuf, 128, 16)
    for ch in range(KCH):
        q = sid + 16 * ch

        @pl.when(q < NCHUNK)
        def _():
            pltpu.sync_copy(zbuf, acc.at[pl.ds(q * 128, 128)])
    plsc.subcore_barrier()

    nb = NBAT // 32
    npairs = nb // 2
    lane = _LANE()

    def start_lin(k, sb, db, eb, sem):
        base = (wid + 32 * k) * B
        pltpu.async_copy(srch.at[pl.ds(base, B)], sb, sem)
        pltpu.async_copy(dsth.at[pl.ds(base, B)], db, sem)
        pltpu.async_copy(eah.at[pl.ds(base, B)], eb, sem)

    def wait_lin(sb, db, eb, sem):
        pltpu.make_async_copy(srch.at[pl.ds(0, B)], sb, sem).wait()
        pltpu.make_async_copy(dsth.at[pl.ds(0, B)], db, sem).wait()
        pltpu.make_async_copy(eah.at[pl.ds(0, B)], eb, sem).wait()

    def do_batch(k, sb, db, eb, asrc, adst, comb):
        base = (wid + 32 * k) * B
        g1 = pltpu.async_copy(adh.at[sb], asrc, gsem)
        g2 = pltpu.async_copy(adh.at[db], adst, gsem)
        g1.wait()
        g2.wait()

        def ebody(e, carry):
            vs = asrc[e]
            vd = adst[e]
            vds = vd.at[(lane + 4) & 15].get(mode="promise_in_bounds")
            t = vs + vds
            ex = jnp.exp(jnp.maximum(t, 0.0) + 0.2 * jnp.minimum(t, 0.0))
            ea = eb[e]
            eash = ea.at[(lane - 4) & 15].get(mode="promise_in_bounds")
            comb[e] = jnp.where(lane < 4, ex,
                                jnp.where(lane < 9, eash, 0.0))
            return carry

        lax.fori_loop(0, B, ebody, 0)
        pltpu.sync_copy(comb, acc.at[db], add=True)
        pltpu.sync_copy(comb, coefh.at[pl.ds(base, B)])

    start_lin(0, sbuf0, dbuf0, eabuf0, lsem0)

    def pair_body(i, carry):
        k0 = 2 * i
        k1 = 2 * i + 1
        wait_lin(sbuf0, dbuf0, eabuf0, lsem0)
        start_lin(k1, sbuf1, dbuf1, eabuf1, lsem1)
        do_batch(k0, sbuf0, dbuf0, eabuf0, adsrc0, addst0, comb0)
        wait_lin(sbuf1, dbuf1, eabuf1, lsem1)

        @pl.when(i + 1 < npairs)
        def _():
            start_lin(k1 + 1, sbuf0, dbuf0, eabuf0, lsem0)

        do_batch(k1, sbuf1, dbuf1, eabuf1, adsrc1, addst1, comb1)
        return carry

    lax.fori_loop(0, npairs, pair_body, 0)

    plsc.subcore_barrier()
    for ch in range(KCH):
        q = sid + 16 * ch

        @pl.when(q < NCHUNK)
        def _():
            r0 = q * 128
            pltpu.sync_copy(acc.at[pl.ds(r0, 128)], bounce)

            @pl.when(cid == 0)
            def _():
                pltpu.sync_copy(bounce, denh.at[0].at[pl.ds(r0, 128)])

            @pl.when(cid == 1)
            def _():
                pltpu.sync_copy(bounce, denh.at[1].at[pl.ds(r0, 128)])


def _s1(srcp, dstp, eap, ad):
    kfn = pl.kernel(
        _s1_kernel, mesh=_MESH,
        compiler_params=pltpu.CompilerParams(use_tc_tiling_on_sc=False),
        out_type=[jax.ShapeDtypeStruct((E2, 16), jnp.float32),
                  jax.ShapeDtypeStruct((2, NR, 16), jnp.float32)],
        scratch_types=[
            pltpu.VMEM((B,), jnp.int32), pltpu.VMEM((B,), jnp.int32),
            pltpu.VMEM((B, 16), jnp.float32),
            pltpu.VMEM((B, 16), jnp.float32), pltpu.VMEM((B, 16), jnp.float32),
            pltpu.VMEM((B, 16), jnp.float32),
            pltpu.VMEM((B,), jnp.int32), pltpu.VMEM((B,), jnp.int32),
            pltpu.VMEM((B, 16), jnp.float32),
            pltpu.VMEM((B, 16), jnp.float32), pltpu.VMEM((B, 16), jnp.float32),
            pltpu.VMEM((B, 16), jnp.float32),
            pltpu.VMEM((128, 16), jnp.float32),
            pltpu.VMEM((128, 16), jnp.float32),
            pltpu.VMEM_SHARED((NR, 16), jnp.float32),
            pltpu.SemaphoreType.DMA, pltpu.SemaphoreType.DMA,
            pltpu.SemaphoreType.DMA,
        ])
    return kfn(srcp, dstp, eap, ad)


def _make_s2(P, C, cmap):
    def s2_kernel(srch, dsth, sch, tph, outh,
                  sbuf0, dbuf0, scbuf0, grows0,
                  sbuf1, dbuf1, scbuf1, grows1,
                  scaled, zbuf, bounce, acc, lsem0, lsem1, gsem):
        cid = lax.axis_index("c")
        sid = lax.axis_index("s")
        nb = NBAT // 16
        _zero_vmem(zbuf, 128, 32)

        def start_lin(k, sb, db, scb, sem):
            base = (sid + 16 * k) * B
            pltpu.async_copy(srch.at[pl.ds(base, B)], sb, sem)
            pltpu.async_copy(dsth.at[pl.ds(base, B)], db, sem)
            pltpu.async_copy(sch.at[pl.ds(base, B)], scb, sem)

        def wait_lin(sb, db, scb, sem):
            pltpu.make_async_copy(srch.at[pl.ds(0, B)], sb, sem).wait()
            pltpu.make_async_copy(dsth.at[pl.ds(0, B)], db, sem).wait()
            pltpu.make_async_copy(sch.at[pl.ds(0, B)], scb, sem).wait()

        for p in range(P):
            pc = 0 if p < P // 2 else 1

            @pl.when(cid == pc)
            def _process():
                for ch in range(KCH):
                    q = sid + 16 * ch

                    @pl.when(q < NCHUNK)
                    def _():
                        pltpu.sync_copy(zbuf, acc.at[pl.ds(q * 128, 128)])
                plsc.subcore_barrier()

                def do_batch(sb, db, scb, grows):
                    pltpu.async_copy(tph.at[p].at[sb], grows, gsem).wait()
                    cidx = jnp.full((16,), cmap[p], jnp.int32)

                    def row_body(r, carry):
                        sv = scb[r].at[cidx].get(mode="promise_in_bounds")
                        for j in range(2):
                            seg = grows[r, pl.ds(16 * j, 16)]
                            scaled[r, pl.ds(16 * j, 16)] = seg * sv
                        return carry

                    lax.fori_loop(0, B, row_body, 0)
                    pltpu.sync_copy(scaled, acc.at[db], add=True)

                npairs = nb // 2
                start_lin(0, sbuf0, dbuf0, scbuf0, lsem0)

                def pair_body(i, carry):
                    k1 = 2 * i + 1
                    wait_lin(sbuf0, dbuf0, scbuf0, lsem0)
                    start_lin(k1, sbuf1, dbuf1, scbuf1, lsem1)
                    do_batch(sbuf0, dbuf0, scbuf0, grows0)
                    wait_lin(sbuf1, dbuf1, scbuf1, lsem1)

                    @pl.when(i + 1 < npairs)
                    def _():
                        start_lin(k1 + 1, sbuf0, dbuf0, scbuf0, lsem0)

                    do_batch(sbuf1, dbuf1, scbuf1, grows1)
                    return carry

                lax.fori_loop(0, npairs, pair_body, 0)

                plsc.subcore_barrier()
                for ch in range(KCH):
                    q = sid + 16 * ch

                    @pl.when(q < NCHUNK)
                    def _():
                        r0 = q * 128
                        pltpu.sync_copy(acc.at[pl.ds(r0, 128)], bounce)
                        pltpu.sync_copy(bounce, outh.at[p].at[pl.ds(r0, 128)])
                plsc.subcore_barrier()

    def run(srcp, dstp, scale, tparts):
        kfn = pl.kernel(
            s2_kernel, mesh=_MESH,
            compiler_params=pltpu.CompilerParams(use_tc_tiling_on_sc=False),
            out_type=[jax.ShapeDtypeStruct((P, NR, 32), jnp.float32)],
            scratch_types=[
                pltpu.VMEM((B,), jnp.int32), pltpu.VMEM((B,), jnp.int32),
                pltpu.VMEM((B, C), jnp.float32),
                pltpu.VMEM((B, 32), jnp.float32),
                pltpu.VMEM((B,), jnp.int32), pltpu.VMEM((B,), jnp.int32),
                pltpu.VMEM((B, C), jnp.float32),
                pltpu.VMEM((B, 32), jnp.float32),
                pltpu.VMEM((B, 32), jnp.float32),
                pltpu.VMEM((128, 32), jnp.float32),
                pltpu.VMEM((128, 32), jnp.float32),
                pltpu.VMEM_SHARED((NR, 32), jnp.float32),
                pltpu.SemaphoreType.DMA, pltpu.SemaphoreType.DMA,
                pltpu.SemaphoreType.DMA,
            ])
        (out,) = kfn(srcp, dstp, scale, tparts)
        return out

    return run


_s2_gat = _make_s2(8, 16, [0, 0, 1, 1, 2, 2, 3, 3])
_s2_gcn = _make_s2(10, 16, [4, 4, 5, 5, 6, 6, 7, 7, 8, 8])


def _s3_kernel(s2h, d2h, hh, g1h, g2h, ibuf, grows, gsem):
    cid = lax.axis_index("c")
    sid = lax.axis_index("s")
    wid = sid * 2 + cid
    for k in range(ETP // (B * 32)):
        base = (wid + 32 * k) * B
        pltpu.sync_copy(s2h.at[pl.ds(base, B)], ibuf)
        pltpu.async_copy(hh.at[ibuf], grows, gsem).wait()
        pltpu.sync_copy(grows, g1h.at[pl.ds(base, B)])
        pltpu.sync_copy(d2h.at[pl.ds(base, B)], ibuf)
        pltpu.async_copy(hh.at[ibuf], grows, gsem).wait()
        pltpu.sync_copy(grows, g2h.at[pl.ds(base, B)])


def _s3(s2p, d2p, h):
    kfn = pl.kernel(
        _s3_kernel, mesh=_MESH,
        compiler_params=pltpu.CompilerParams(use_tc_tiling_on_sc=False),
        out_type=[jax.ShapeDtypeStruct((ETP, 320), jnp.float32),
                  jax.ShapeDtypeStruct((ETP, 320), jnp.float32)],
        scratch_types=[
            pltpu.VMEM((B,), jnp.int32),
            pltpu.VMEM((B, 320), jnp.float32),
            pltpu.SemaphoreType.DMA,
        ])
    return kfn(s2p, d2p, h)


def kernel(x, edge_index, edge_attr, edge_index_test, Wg, att_src, att_dst, bg,
           W_gcnA, b_gcnA, W_gcnB, b_gcnB,
           W_l0, b_l0, W_l1, b_l1, W_l2, b_l2, W_l3, b_l3, W_l4, b_l4):
    f32 = jnp.float32
    src, dst = edge_index[0], edge_index[1]
    npad = E2 - E
    srcp = jnp.concatenate([src, jnp.zeros((npad,), src.dtype)])
    dstp = jnp.concatenate([dst, jnp.full((npad,), N, dst.dtype)])
    eap = jnp.concatenate([edge_attr, jnp.zeros((npad, NEA), f32)], axis=0)
    eap = jnp.pad(eap, ((0, 0), (0, 16 - NEA)))

    xp = jnp.pad(x, ((0, 0), (0, 128 - DIN)))
    wgp = jnp.pad(Wg, ((0, 128 - DIN), (0, 0)))
    atts = att_src.reshape(1, H * HID)
    attd = att_dst.reshape(1, H * HID)

    hp_parts, ad = _m1(xp, wgp, atts, attd)
    adp = jnp.concatenate([ad, jnp.zeros((NPAD, 16), f32)], axis=0)

    coef, den2 = _s1(srcp, dstp, eap, adp)

    num_parts = _s2_gat(srcp, dstp, coef, hp_parts)

    wcat_a = jnp.transpose(W_gcnA, (1, 0, 2)).reshape(H * HID, NEA * HID)
    hp, dv = _gatepi(num_parts, hp_parts, ad, den2, bg.reshape(1, -1), wcat_a)

    for l in range(4):
        s_parts = _s2_gcn(srcp, dstp, coef, hp)
        bl = (b_gcnA if l == 0 else b_gcnB[l - 1]).reshape(1, -1)
        wcat = jnp.transpose(W_gcnB[l], (1, 0, 2)).reshape(NEA * HID, NEA * HID)
        hp = _layer(s_parts, hp, dv, bl, wcat)

    s_parts = _s2_gcn(srcp, dstp, coef, hp)
    h_final = _final(s_parts, hp, dv, b_gcnB[3].reshape(1, -1))

    s2i, d2i = edge_index_test[0], edge_index_test[1]
    tpad = ETP - ET
    s2p = jnp.concatenate([s2i, jnp.zeros((tpad,), s2i.dtype)])
    d2p = jnp.concatenate([d2i, jnp.zeros((tpad,), d2i.dtype)])
    g1, g2 = _s3(s2p, d2p, h_final)

    out = _mlp(g1, g2,
               W_l0, b_l0.reshape(1, -1), W_l1, b_l1.reshape(1, -1),
               W_l2, b_l2.reshape(1, -1), W_l3, b_l3.reshape(1, -1),
               W_l4, b_l4.reshape(1, -1))
    return out[:ET]


# speedup vs baseline: 9.5269x; 1.0523x over previous
import functools

import jax
import jax.numpy as jnp
from jax import lax
from jax.experimental import pallas as pl
from jax.experimental.pallas import tpu as pltpu
from jax.experimental.pallas import tpu_sc as plsc

N = 50000
E = 800000
ET = 100000
DIN = 74
HID = 64
H = 4
NEA = 5
HL = 128
OUT = 4

BLK = 1000
NBLK = N // BLK
B = 128
E2 = 811008
NBAT = E2 // B
ETP = 102400
NPAD = 16
NR = 50048
NCHUNK = NR // 128
KCH = 25


def _m1_body(xp_ref, wg_ref, atts_ref, attd_ref, hp_ref, ad_ref):
    h = jnp.dot(xp_ref[...], wg_ref[...], preferred_element_type=jnp.float32)
    ps = h * atts_ref[...]
    pd = h * attd_ref[...]
    cols = []
    for hd in range(H):
        cols.append(jnp.sum(ps[:, 64 * hd:64 * hd + 64], axis=1, keepdims=True))
    for hd in range(H):
        cols.append(jnp.sum(pd[:, 64 * hd:64 * hd + 64], axis=1, keepdims=True))
    a = jnp.concatenate(cols, axis=1)
    pad = jnp.zeros_like(a)
    ad_ref[...] = jnp.concatenate([a, pad], axis=1)
    for p in range(8):
        hp_ref[p] = h[:, 32 * p:32 * p + 32]


def _m1(xp, wgp, atts, attd):
    return pl.pallas_call(
        _m1_body,
        grid=(NBLK,),
        in_specs=[pl.BlockSpec((BLK, 128), lambda i: (i, 0)),
                  pl.BlockSpec((128, 256), lambda i: (0, 0)),
                  pl.BlockSpec((1, 256), lambda i: (0, 0)),
                  pl.BlockSpec((1, 256), lambda i: (0, 0))],
        out_specs=[pl.BlockSpec((8, BLK, 32), lambda i: (0, i, 0)),
                   pl.BlockSpec((BLK, 16), lambda i: (i, 0))],
        out_shape=[jax.ShapeDtypeStruct((8, N, 32), jnp.float32),
                   jax.ShapeDtypeStruct((N, 16), jnp.float32)],
    )(xp, wgp, atts, attd)


def _gatepi_body(num_ref, hp_ref, ad_ref, den_ref, bg_ref, wcat_ref,
                 hpa_ref, dv_ref):
    ad = ad_ref[...]
    dent = den_ref[0] + den_ref[1]
    a_sum = ad[:, 0:4] + ad[:, 4:8]
    lk = jnp.maximum(a_sum, 0.0) + 0.2 * jnp.minimum(a_sum, 0.0)
    exself = jnp.exp(lk)
    den = dent[:, 0:4] + exself + 1e-16
    deg = dent[:, 4:9] + 1.0
    dv = lax.rsqrt(deg)
    segs = []
    for p in range(8):
        hd = p // 2
        t = (num_ref[p] + exself[:, hd:hd + 1] * hp_ref[p]) / den[:, hd:hd + 1]
        segs.append(t)
    h0 = jnp.concatenate(segs, axis=1) + bg_ref[...]
    h0 = jnp.maximum(h0, 0.0)
    hpa = jnp.dot(h0, wcat_ref[...], preferred_element_type=jnp.float32)
    for p in range(10):
        i = p // 2
        hpa_ref[p] = hpa[:, 32 * p:32 * p + 32] * dv[:, i:i + 1]
    dv_ref[...] = jnp.concatenate([dv, jnp.zeros_like(dv[:, 0:3])], axis=1)


def _gatepi(num_parts, hp_parts, ad, den2, bg2, wcat_a):
    return pl.pallas_call(
        _gatepi_body,
        grid=(NBLK,),
        in_specs=[pl.BlockSpec((8, BLK, 32), lambda i: (0, i, 0)),
                  pl.BlockSpec((8, BLK, 32), lambda i: (0, i, 0)),
                  pl.BlockSpec((BLK, 16), lambda i: (i, 0)),
                  pl.BlockSpec((2, BLK, 16), lambda i: (0, i, 0)),
                  pl.BlockSpec((1, 256), lambda i: (0, 0)),
                  pl.BlockSpec((256, 320), lambda i: (0, 0))],
        out_specs=[pl.BlockSpec((10, BLK, 32), lambda i: (0, i, 0)),
                   pl.BlockSpec((BLK, 8), lambda i: (i, 0))],
        out_shape=[jax.ShapeDtypeStruct((10, N, 32), jnp.float32),
                   jax.ShapeDtypeStruct((N, 8), jnp.float32)],
    )(num_parts, hp_parts, ad, den2, bg2, wcat_a)


def _layer_body(s_ref, hp_ref, dv_ref, b_ref, wcat_ref, out_ref):
    dv = dv_ref[...]
    segs = []
    for p in range(10):
        i = p // 2
        t = dv[:, i:i + 1] * (s_ref[p] + hp_ref[p]) + b_ref[0, 32 * p:32 * p + 32]
        segs.append(t)
    h = jnp.maximum(jnp.concatenate(segs, axis=1), 0.0)
    hn = jnp.dot(h, wcat_ref[...], preferred_element_type=jnp.float32)
    for p in range(10):
        i = p // 2
        out_ref[p] = hn[:, 32 * p:32 * p + 32] * dv[:, i:i + 1]


def _layer(s_parts, hp_parts, dv, b2, wcat):
    return pl.pallas_call(
        _layer_body,
        grid=(NBLK,),
        in_specs=[pl.BlockSpec((10, BLK, 32), lambda i: (0, i, 0)),
                  pl.BlockSpec((10, BLK, 32), lambda i: (0, i, 0)),
                  pl.BlockSpec((BLK, 8), lambda i: (i, 0)),
                  pl.BlockSpec((1, 320), lambda i: (0, 0)),
                  pl.BlockSpec((320, 320), lambda i: (0, 0))],
        out_specs=pl.BlockSpec((10, BLK, 32), lambda i: (0, i, 0)),
        out_shape=jax.ShapeDtypeStruct((10, N, 32), jnp.float32),
    )(s_parts, hp_parts, dv, b2, wcat)


def _final_body(s_ref, hp_ref, dv_ref, b_ref, out_ref):
    dv = dv_ref[...]
    segs = []
    for p in range(10):
        i = p // 2
        t = dv[:, i:i + 1] * (s_ref[p] + hp_ref[p]) + b_ref[0, 32 * p:32 * p + 32]
        segs.append(t)
    out_ref[...] = jnp.maximum(jnp.concatenate(segs, axis=1), 0.0)


def _final(s_parts, hp_parts, dv, b2):
    return pl.pallas_call(
        _final_body,
        grid=(NBLK,),
        in_specs=[pl.BlockSpec((10, BLK, 32), lambda i: (0, i, 0)),
                  pl.BlockSpec((10, BLK, 32), lambda i: (0, i, 0)),
                  pl.BlockSpec((BLK, 8), lambda i: (i, 0)),
                  pl.BlockSpec((1, 320), lambda i: (0, 0))],
        out_specs=pl.BlockSpec((BLK, 320), lambda i: (i, 0)),
        out_shape=jax.ShapeDtypeStruct((N, 320), jnp.float32),
    )(s_parts, hp_parts, dv, b2)


def _mlp_body(g1_ref, g2_ref, w0_ref, b0_ref, w1_ref, b1_ref, w2_ref, b2_ref,
              w3_ref, b3_ref, w4_ref, b4_ref, o_ref):
    def mlp(e):
        z = jnp.maximum(
            jnp.dot(e, w0_ref[...], preferred_element_type=jnp.float32)
            + b0_ref[...], 0.0)
        z = jnp.maximum(
            jnp.dot(z, w1_ref[...], preferred_element_type=jnp.float32)
            + b1_ref[...], 0.0)
        z = jnp.dot(z, w2_ref[...], preferred_element_type=jnp.float32) + b2_ref[...]
        z = jnp.maximum(
            jnp.dot(z, w3_ref[...], preferred_element_type=jnp.float32)
            + b3_ref[...], 0.0)
        return jnp.dot(z, w4_ref[...], preferred_element_type=jnp.float32) + b4_ref[...]

    g1 = g1_ref[...]
    g2 = g2_ref[...]
    t1 = mlp(jnp.concatenate([g1, g2], axis=1))
    t2 = mlp(jnp.concatenate([g2, g1], axis=1))
    t2p = jnp.concatenate(
        [t2[:, 0:1], t2[:, 2:3], t2[:, 1:2], t2[:, 3:4]], axis=1)
    o_ref[...] = 0.5 * (t1 + t2p)


def _mlp(g1, g2, w0, b0, w1, b1, w2, b2, w3, b3, w4, b4):
    mblk = 2048
    spec_g = pl.BlockSpec((mblk, 320), lambda i: (i, 0))
    full = lambda *s: pl.BlockSpec(s, lambda i: (0,) * len(s))
    return pl.pallas_call(
        _mlp_body,
        grid=(ETP // mblk,),
        in_specs=[spec_g, spec_g,
                  full(640, HL), full(1, HL), full(HL, HL), full(1, HL),
                  full(HL, HL), full(1, HL), full(HL, HL), full(1, HL),
                  full(HL, OUT), full(1, OUT)],
        out_specs=pl.BlockSpec((mblk, OUT), lambda i: (i, 0)),
        out_shape=jax.ShapeDtypeStruct((ETP, OUT), jnp.float32),
    )(g1, g2, w0, b0, w1, b1, w2, b2, w3, b3, w4, b4)


_MESH = plsc.VectorSubcoreMesh(core_axis_name="c", subcore_axis_name="s")
_LANE = lambda: lax.broadcasted_iota(jnp.int32, (16,), 0)


def _zero_vmem(ref, rows, cols):
    z = jnp.zeros((16,), jnp.float32)
    for r in range(rows):
        for j in range(cols // 16):
            ref[r, pl.ds(16 * j, 16)] = z


def _s1_kernel(srch, dsth, eah, adh, coefh, denh,
               sbuf0, dbuf0, eabuf0, adsrc0, addst0, comb0,
               sbuf1, dbuf1, eabuf1, adsrc1, addst1, comb1,
               zbuf, bounce, acc, lsem0, lsem1, gsem):
    cid = lax.axis_index("c")
    sid = lax.axis_index("s")
    wid = sid * 2 + cid

    _zero_vmem(zbuf, 128, 16)
    for ch in range(KCH):
        q = sid + 16 * ch

        @pl.when(q < NCHUNK)
        def _():
            pltpu.sync_copy(zbuf, acc.at[pl.ds(q * 128, 128)])
    plsc.subcore_barrier()

    nb = NBAT // 32
    npairs = nb // 2
    lane = _LANE()

    def start_lin(k, sb, db, eb, sem):
        base = (wid + 32 * k) * B
        pltpu.async_copy(srch.at[pl.ds(base, B)], sb, sem)
        pltpu.async_copy(dsth.at[pl.ds(base, B)], db, sem)
        pltpu.async_copy(eah.at[pl.ds(base, B)], eb, sem)

    def wait_lin(sb, db, eb, sem):
        pltpu.make_async_copy(srch.at[pl.ds(0, B)], sb, sem).wait()
        pltpu.make_async_copy(dsth.at[pl.ds(0, B)], db, sem).wait()
        pltpu.make_async_copy(eah.at[pl.ds(0, B)], eb, sem).wait()

    def do_batch(k, sb, db, eb, asrc, adst, comb):
        base = (wid + 32 * k) * B
        g1 = pltpu.async_copy(adh.at[sb], asrc, gsem)
        g2 = pltpu.async_copy(adh.at[db], adst, gsem)
        g1.wait()
        g2.wait()

        def ebody(e, carry):
            vs = asrc[e]
            vd = adst[e]
            vds = vd.at[(lane + 4) & 15].get(mode="promise_in_bounds")
            t = vs + vds
            ex = jnp.exp(jnp.maximum(t, 0.0) + 0.2 * jnp.minimum(t, 0.0))
            ea = eb[e]
            eash = ea.at[(lane - 4) & 15].get(mode="promise_in_bounds")
            comb[e] = jnp.where(lane < 4, ex,
                                jnp.where(lane < 9, eash, 0.0))
            return carry

        lax.fori_loop(0, B, ebody, 0)
        pltpu.sync_copy(comb, acc.at[db], add=True)
        pltpu.sync_copy(comb, coefh.at[pl.ds(base, B)])

    start_lin(0, sbuf0, dbuf0, eabuf0, lsem0)

    def pair_body(i, carry):
        k0 = 2 * i
        k1 = 2 * i + 1
        wait_lin(sbuf0, dbuf0, eabuf0, lsem0)
        start_lin(k1, sbuf1, dbuf1, eabuf1, lsem1)
        do_batch(k0, sbuf0, dbuf0, eabuf0, adsrc0, addst0, comb0)
        wait_lin(sbuf1, dbuf1, eabuf1, lsem1)

        @pl.when(i + 1 < npairs)
        def _():
            start_lin(k1 + 1, sbuf0, dbuf0, eabuf0, lsem0)

        do_batch(k1, sbuf1, dbuf1, eabuf1, adsrc1, addst1, comb1)
        return carry

    lax.fori_loop(0, npairs, pair_body, 0)

    plsc.subcore_barrier()
    for ch in range(KCH):
        q = sid + 16 * ch

        @pl.when(q < NCHUNK)
        def _():
            r0 = q * 128
            pltpu.sync_copy(acc.at[pl.ds(r0, 128)], bounce)

            @pl.when(cid == 0)
            def _():
                pltpu.sync_copy(bounce, denh.at[0].at[pl.ds(r0, 128)])

            @pl.when(cid == 1)
            def _():
                pltpu.sync_copy(bounce, denh.at[1].at[pl.ds(r0, 128)])


def _s1(srcp, dstp, eap, ad):
    kfn = pl.kernel(
        _s1_kernel, mesh=_MESH,
        compiler_params=pltpu.CompilerParams(use_tc_tiling_on_sc=False),
        out_type=[jax.ShapeDtypeStruct((E2, 16), jnp.float32),
                  jax.ShapeDtypeStruct((2, NR, 16), jnp.float32)],
        scratch_types=[
            pltpu.VMEM((B,), jnp.int32), pltpu.VMEM((B,), jnp.int32),
            pltpu.VMEM((B, 16), jnp.float32),
            pltpu.VMEM((B, 16), jnp.float32), pltpu.VMEM((B, 16), jnp.float32),
            pltpu.VMEM((B, 16), jnp.float32),
            pltpu.VMEM((B,), jnp.int32), pltpu.VMEM((B,), jnp.int32),
            pltpu.VMEM((B, 16), jnp.float32),
            pltpu.VMEM((B, 16), jnp.float32), pltpu.VMEM((B, 16), jnp.float32),
            pltpu.VMEM((B, 16), jnp.float32),
            pltpu.VMEM((128, 16), jnp.float32),
            pltpu.VMEM((128, 16), jnp.float32),
            pltpu.VMEM_SHARED((NR, 16), jnp.float32),
            pltpu.SemaphoreType.DMA, pltpu.SemaphoreType.DMA,
            pltpu.SemaphoreType.DMA,
        ])
    return kfn(srcp, dstp, eap, ad)


def _make_s2(P, C, cmap):
    def s2_kernel(srch, dsth, sch, tph, outh,
                  sbuf0, dbuf0, scbuf0, grows0,
                  sbuf1, dbuf1, scbuf1, grows1,
                  scaled, zbuf, bounce, acc, lsem0, lsem1, gsem):
        cid = lax.axis_index("c")
        sid = lax.axis_index("s")
        nb = NBAT // 16
        _zero_vmem(zbuf, 128, 32)

        def start_lin(k, sb, db, scb, sem):
            base = (sid + 16 * k) * B
            pltpu.async_copy(srch.at[pl.ds(base, B)], sb, sem)
            pltpu.async_copy(dsth.at[pl.ds(base, B)], db, sem)
            pltpu.async_copy(sch.at[pl.ds(base, B)], scb, sem)

        def wait_lin(sb, db, scb, sem):
            pltpu.make_async_copy(srch.at[pl.ds(0, B)], sb, sem).wait()
            pltpu.make_async_copy(dsth.at[pl.ds(0, B)], db, sem).wait()
            pltpu.make_async_copy(sch.at[pl.ds(0, B)], scb, sem).wait()

        for p in range(P):
            pc = 0 if p < P // 2 else 1

            @pl.when(cid == pc)
            def _process():
                for ch in range(KCH):
                    q = sid + 16 * ch

                    @pl.when(q < NCHUNK)
                    def _():
                        pltpu.sync_copy(zbuf, acc.at[pl.ds(q * 128, 128)])
                plsc.subcore_barrier()

                cidx = jnp.full((16,), cmap[p], jnp.int32)

                def start_gather(sb, grows):
                    pltpu.async_copy(tph.at[p].at[sb], grows, gsem)

                def wait_gather(sb, grows):
                    pltpu.make_async_copy(
                        tph.at[p].at[sb], grows, gsem).wait()

                def compute_batch(db, scb, grows):
                    def row_body(rr, carry):
                        for u in range(4):
                            r = rr * 4 + u
                            sv = scb[r].at[cidx].get(mode="promise_in_bounds")
                            for j in range(2):
                                seg = grows[r, pl.ds(16 * j, 16)]
                                scaled[r, pl.ds(16 * j, 16)] = seg * sv
                        return carry

                    lax.fori_loop(0, B // 4, row_body, 0)
                    pltpu.sync_copy(scaled, acc.at[db], add=True)

                npairs = nb // 2
                start_lin(0, sbuf0, dbuf0, scbuf0, lsem0)
                wait_lin(sbuf0, dbuf0, scbuf0, lsem0)
                start_gather(sbuf0, grows0)
                start_lin(1, sbuf1, dbuf1, scbuf1, lsem1)

                def pair_body(i, carry):
                    k1 = 2 * i + 1
                    wait_lin(sbuf1, dbuf1, scbuf1, lsem1)
                    start_gather(sbuf1, grows1)
                    wait_gather(sbuf0, grows0)
                    compute_batch(dbuf0, scbuf0, grows0)

                    @pl.when(i + 1 < npairs)
                    def _():
                        start_lin(k1 + 1, sbuf0, dbuf0, scbuf0, lsem0)
                        wait_lin(sbuf0, dbuf0, scbuf0, lsem0)
                        start_gather(sbuf0, grows0)

                    wait_gather(sbuf1, grows1)
                    compute_batch(dbuf1, scbuf1, grows1)

                    @pl.when(i + 1 < npairs)
                    def _():
                        start_lin(k1 + 2, sbuf1, dbuf1, scbuf1, lsem1)

                    return carry

                lax.fori_loop(0, npairs, pair_body, 0)

                plsc.subcore_barrier()
                for ch in range(KCH):
                    q = sid + 16 * ch

                    @pl.when(q < NCHUNK)
                    def _():
                        r0 = q * 128
                        pltpu.sync_copy(acc.at[pl.ds(r0, 128)], bounce)
                        pltpu.sync_copy(bounce, outh.at[p].at[pl.ds(r0, 128)])
                plsc.subcore_barrier()

    def run(srcp, dstp, scale, tparts):
        kfn = pl.kernel(
            s2_kernel, mesh=_MESH,
            compiler_params=pltpu.CompilerParams(use_tc_tiling_on_sc=False),
            out_type=[jax.ShapeDtypeStruct((P, NR, 32), jnp.float32)],
            scratch_types=[
                pltpu.VMEM((B,), jnp.int32), pltpu.VMEM((B,), jnp.int32),
                pltpu.VMEM((B, C), jnp.float32),
                pltpu.VMEM((B, 32), jnp.float32),
                pltpu.VMEM((B,), jnp.int32), pltpu.VMEM((B,), jnp.int32),
                pltpu.VMEM((B, C), jnp.float32),
                pltpu.VMEM((B, 32), jnp.float32),
                pltpu.VMEM((B, 32), jnp.float32),
                pltpu.VMEM((128, 32), jnp.float32),
                pltpu.VMEM((128, 32), jnp.float32),
                pltpu.VMEM_SHARED((NR, 32), jnp.float32),
                pltpu.SemaphoreType.DMA, pltpu.SemaphoreType.DMA,
                pltpu.SemaphoreType.DMA,
            ])
        (out,) = kfn(srcp, dstp, scale, tparts)
        return out

    return run


_s2_gat = _make_s2(8, 16, [0, 0, 1, 1, 2, 2, 3, 3])
_s2_gcn = _make_s2(10, 16, [4, 4, 5, 5, 6, 6, 7, 7, 8, 8])


def _s3_kernel(s2h, d2h, hh, g1h, g2h, ibuf, grows, gsem):
    cid = lax.axis_index("c")
    sid = lax.axis_index("s")
    wid = sid * 2 + cid
    for k in range(ETP // (B * 32)):
        base = (wid + 32 * k) * B
        pltpu.sync_copy(s2h.at[pl.ds(base, B)], ibuf)
        pltpu.async_copy(hh.at[ibuf], grows, gsem).wait()
        pltpu.sync_copy(grows, g1h.at[pl.ds(base, B)])
        pltpu.sync_copy(d2h.at[pl.ds(base, B)], ibuf)
        pltpu.async_copy(hh.at[ibuf], grows, gsem).wait()
        pltpu.sync_copy(grows, g2h.at[pl.ds(base, B)])


def _s3(s2p, d2p, h):
    kfn = pl.kernel(
        _s3_kernel, mesh=_MESH,
        compiler_params=pltpu.CompilerParams(use_tc_tiling_on_sc=False),
        out_type=[jax.ShapeDtypeStruct((ETP, 320), jnp.float32),
                  jax.ShapeDtypeStruct((ETP, 320), jnp.float32)],
        scratch_types=[
            pltpu.VMEM((B,), jnp.int32),
            pltpu.VMEM((B, 320), jnp.float32),
            pltpu.SemaphoreType.DMA,
        ])
    return kfn(s2p, d2p, h)


def kernel(x, edge_index, edge_attr, edge_index_test, Wg, att_src, att_dst, bg,
           W_gcnA, b_gcnA, W_gcnB, b_gcnB,
           W_l0, b_l0, W_l1, b_l1, W_l2, b_l2, W_l3, b_l3, W_l4, b_l4):
    f32 = jnp.float32
    src, dst = edge_index[0], edge_index[1]
    npad = E2 - E
    srcp = jnp.concatenate([src, jnp.zeros((npad,), src.dtype)])
    dstp = jnp.concatenate([dst, jnp.full((npad,), N, dst.dtype)])
    eap = jnp.concatenate([edge_attr, jnp.zeros((npad, NEA), f32)], axis=0)
    eap = jnp.pad(eap, ((0, 0), (0, 16 - NEA)))

    xp = jnp.pad(x, ((0, 0), (0, 128 - DIN)))
    wgp = jnp.pad(Wg, ((0, 128 - DIN), (0, 0)))
    atts = att_src.reshape(1, H * HID)
    attd = att_dst.reshape(1, H * HID)

    hp_parts, ad = _m1(xp, wgp, atts, attd)
    adp = jnp.concatenate([ad, jnp.zeros((NPAD, 16), f32)], axis=0)

    coef, den2 = _s1(srcp, dstp, eap, adp)

    num_parts = _s2_gat(srcp, dstp, coef, hp_parts)

    wcat_a = jnp.transpose(W_gcnA, (1, 0, 2)).reshape(H * HID, NEA * HID)
    hp, dv = _gatepi(num_parts, hp_parts, ad, den2, bg.reshape(1, -1), wcat_a)

    for l in range(4):
        s_parts = _s2_gcn(srcp, dstp, coef, hp)
        bl = (b_gcnA if l == 0 else b_gcnB[l - 1]).reshape(1, -1)
        wcat = jnp.transpose(W_gcnB[l], (1, 0, 2)).reshape(NEA * HID, NEA * HID)
        hp = _layer(s_parts, hp, dv, bl, wcat)

    s_parts = _s2_gcn(srcp, dstp, coef, hp)
    h_final = _final(s_parts, hp, dv, b_gcnB[3].reshape(1, -1))

    s2i, d2i = edge_index_test[0], edge_index_test[1]
    tpad = ETP - ET
    s2p = jnp.concatenate([s2i, jnp.zeros((tpad,), s2i.dtype)])
    d2p = jnp.concatenate([d2i, jnp.zeros((tpad,), d2i.dtype)])
    g1, g2 = _s3(s2p, d2p, h_final)

    out = _mlp(g1, g2,
               W_l0, b_l0.reshape(1, -1), W_l1, b_l1.reshape(1, -1),
               W_l2, b_l2.reshape(1, -1), W_l3, b_l3.reshape(1, -1),
               W_l4, b_l4.reshape(1, -1))
    return out[:ET]


# speedup vs baseline: 12.2643x; 1.2873x over previous
import functools

import jax
import jax.numpy as jnp
from jax import lax
from jax.experimental import pallas as pl
from jax.experimental.pallas import tpu as pltpu
from jax.experimental.pallas import tpu_sc as plsc

N = 50000
E = 800000
ET = 100000
DIN = 74
HID = 64
H = 4
NEA = 5
HL = 128
OUT = 4

BLK = 1000
NBLK = N // BLK
B = 128
E2 = 811008
NBAT = E2 // B
ETP = 102400
NPAD = 16
NR = 50048
NCHUNK = NR // 128
KCH = 25


def _m1_body(xp_ref, wg_ref, atts_ref, attd_ref, hp_ref, ad_ref):
    h = jnp.dot(xp_ref[...], wg_ref[...], preferred_element_type=jnp.float32)
    ps = h * atts_ref[...]
    pd = h * attd_ref[...]
    cols = []
    for hd in range(H):
        cols.append(jnp.sum(ps[:, 64 * hd:64 * hd + 64], axis=1, keepdims=True))
    for hd in range(H):
        cols.append(jnp.sum(pd[:, 64 * hd:64 * hd + 64], axis=1, keepdims=True))
    a = jnp.concatenate(cols, axis=1)
    pad = jnp.zeros_like(a)
    ad_ref[...] = jnp.concatenate([a, pad], axis=1)
    for p in range(8):
        hp_ref[p] = h[:, 32 * p:32 * p + 32]


def _m1(xp, wgp, atts, attd):
    return pl.pallas_call(
        _m1_body,
        grid=(NBLK,),
        in_specs=[pl.BlockSpec((BLK, 128), lambda i: (i, 0)),
                  pl.BlockSpec((128, 256), lambda i: (0, 0)),
                  pl.BlockSpec((1, 256), lambda i: (0, 0)),
                  pl.BlockSpec((1, 256), lambda i: (0, 0))],
        out_specs=[pl.BlockSpec((8, BLK, 32), lambda i: (0, i, 0)),
                   pl.BlockSpec((BLK, 16), lambda i: (i, 0))],
        out_shape=[jax.ShapeDtypeStruct((8, N, 32), jnp.float32),
                   jax.ShapeDtypeStruct((N, 16), jnp.float32)],
    )(xp, wgp, atts, attd)


def _gatepi_body(num_ref, hp_ref, ad_ref, den_ref, bg_ref, wcat_ref,
                 hpa_ref, dv_ref):
    ad = ad_ref[...]
    dent = den_ref[0] + den_ref[1]
    a_sum = ad[:, 0:4] + ad[:, 4:8]
    lk = jnp.maximum(a_sum, 0.0) + 0.2 * jnp.minimum(a_sum, 0.0)
    exself = jnp.exp(lk)
    den = dent[:, 0:4] + exself + 1e-16
    deg = dent[:, 4:9] + 1.0
    dv = lax.rsqrt(deg)
    segs = []
    for p in range(8):
        hd = p // 2
        t = (num_ref[p] + exself[:, hd:hd + 1] * hp_ref[p]) / den[:, hd:hd + 1]
        segs.append(t)
    h0 = jnp.concatenate(segs, axis=1) + bg_ref[...]
    h0 = jnp.maximum(h0, 0.0)
    hpa = jnp.dot(h0, wcat_ref[...], preferred_element_type=jnp.float32)
    for p in range(10):
        i = p // 2
        hpa_ref[p] = hpa[:, 32 * p:32 * p + 32] * dv[:, i:i + 1]
    dv_ref[...] = jnp.concatenate([dv, jnp.zeros_like(dv[:, 0:3])], axis=1)


def _gatepi(num_parts, hp_parts, ad, den2, bg2, wcat_a):
    return pl.pallas_call(
        _gatepi_body,
        grid=(NBLK,),
        in_specs=[pl.BlockSpec((8, BLK, 32), lambda i: (0, i, 0)),
                  pl.BlockSpec((8, BLK, 32), lambda i: (0, i, 0)),
                  pl.BlockSpec((BLK, 16), lambda i: (i, 0)),
                  pl.BlockSpec((2, BLK, 16), lambda i: (0, i, 0)),
                  pl.BlockSpec((1, 256), lambda i: (0, 0)),
                  pl.BlockSpec((256, 320), lambda i: (0, 0))],
        out_specs=[pl.BlockSpec((10, BLK, 32), lambda i: (0, i, 0)),
                   pl.BlockSpec((BLK, 8), lambda i: (i, 0))],
        out_shape=[jax.ShapeDtypeStruct((10, N, 32), jnp.float32),
                   jax.ShapeDtypeStruct((N, 8), jnp.float32)],
    )(num_parts, hp_parts, ad, den2, bg2, wcat_a)


def _layer_body(s_ref, hp_ref, dv_ref, b_ref, wcat_ref, out_ref):
    dv = dv_ref[...]
    segs = []
    for p in range(10):
        i = p // 2
        t = dv[:, i:i + 1] * (s_ref[p] + hp_ref[p]) + b_ref[0, 32 * p:32 * p + 32]
        segs.append(t)
    h = jnp.maximum(jnp.concatenate(segs, axis=1), 0.0)
    hn = jnp.dot(h, wcat_ref[...], preferred_element_type=jnp.float32)
    for p in range(10):
        i = p // 2
        out_ref[p] = hn[:, 32 * p:32 * p + 32] * dv[:, i:i + 1]


def _layer(s_parts, hp_parts, dv, b2, wcat):
    return pl.pallas_call(
        _layer_body,
        grid=(NBLK,),
        in_specs=[pl.BlockSpec((10, BLK, 32), lambda i: (0, i, 0)),
                  pl.BlockSpec((10, BLK, 32), lambda i: (0, i, 0)),
                  pl.BlockSpec((BLK, 8), lambda i: (i, 0)),
                  pl.BlockSpec((1, 320), lambda i: (0, 0)),
                  pl.BlockSpec((320, 320), lambda i: (0, 0))],
        out_specs=pl.BlockSpec((10, BLK, 32), lambda i: (0, i, 0)),
        out_shape=jax.ShapeDtypeStruct((10, N, 32), jnp.float32),
    )(s_parts, hp_parts, dv, b2, wcat)


def _final_body(s_ref, hp_ref, dv_ref, b_ref, out_ref):
    dv = dv_ref[...]
    segs = []
    for p in range(10):
        i = p // 2
        t = dv[:, i:i + 1] * (s_ref[p] + hp_ref[p]) + b_ref[0, 32 * p:32 * p + 32]
        segs.append(t)
    out_ref[...] = jnp.maximum(jnp.concatenate(segs, axis=1), 0.0)


def _final(s_parts, hp_parts, dv, b2):
    return pl.pallas_call(
        _final_body,
        grid=(NBLK,),
        in_specs=[pl.BlockSpec((10, BLK, 32), lambda i: (0, i, 0)),
                  pl.BlockSpec((10, BLK, 32), lambda i: (0, i, 0)),
                  pl.BlockSpec((BLK, 8), lambda i: (i, 0)),
                  pl.BlockSpec((1, 320), lambda i: (0, 0))],
        out_specs=pl.BlockSpec((BLK, 320), lambda i: (i, 0)),
        out_shape=jax.ShapeDtypeStruct((N, 320), jnp.float32),
    )(s_parts, hp_parts, dv, b2)


def _mlp_body(g1_ref, g2_ref, w0_ref, b0_ref, w1_ref, b1_ref, w2_ref, b2_ref,
              w3_ref, b3_ref, w4_ref, b4_ref, o_ref):
    def mlp(e):
        z = jnp.maximum(
            jnp.dot(e, w0_ref[...], preferred_element_type=jnp.float32)
            + b0_ref[...], 0.0)
        z = jnp.maximum(
            jnp.dot(z, w1_ref[...], preferred_element_type=jnp.float32)
            + b1_ref[...], 0.0)
        z = jnp.dot(z, w2_ref[...], preferred_element_type=jnp.float32) + b2_ref[...]
        z = jnp.maximum(
            jnp.dot(z, w3_ref[...], preferred_element_type=jnp.float32)
            + b3_ref[...], 0.0)
        return jnp.dot(z, w4_ref[...], preferred_element_type=jnp.float32) + b4_ref[...]

    g1 = g1_ref[...]
    g2 = g2_ref[...]
    t1 = mlp(jnp.concatenate([g1, g2], axis=1))
    t2 = mlp(jnp.concatenate([g2, g1], axis=1))
    t2p = jnp.concatenate(
        [t2[:, 0:1], t2[:, 2:3], t2[:, 1:2], t2[:, 3:4]], axis=1)
    o_ref[...] = 0.5 * (t1 + t2p)


def _mlp(g1, g2, w0, b0, w1, b1, w2, b2, w3, b3, w4, b4):
    mblk = 2048
    spec_g = pl.BlockSpec((mblk, 320), lambda i: (i, 0))
    full = lambda *s: pl.BlockSpec(s, lambda i: (0,) * len(s))
    return pl.pallas_call(
        _mlp_body,
        grid=(ETP // mblk,),
        in_specs=[spec_g, spec_g,
                  full(640, HL), full(1, HL), full(HL, HL), full(1, HL),
                  full(HL, HL), full(1, HL), full(HL, HL), full(1, HL),
                  full(HL, OUT), full(1, OUT)],
        out_specs=pl.BlockSpec((mblk, OUT), lambda i: (i, 0)),
        out_shape=jax.ShapeDtypeStruct((ETP, OUT), jnp.float32),
    )(g1, g2, w0, b0, w1, b1, w2, b2, w3, b3, w4, b4)


_MESH = plsc.VectorSubcoreMesh(core_axis_name="c", subcore_axis_name="s")
_LANE = lambda: lax.broadcasted_iota(jnp.int32, (16,), 0)


def _zero_vmem(ref, rows, cols):
    z = jnp.zeros((16,), jnp.float32)
    for r in range(rows):
        for j in range(cols // 16):
            ref[r, pl.ds(16 * j, 16)] = z


def _s1_kernel(srch, dsth, eah, adh, coefh, denh,
               sbuf0, dbuf0, eabuf0, adsrc0, addst0, comb0,
               sbuf1, dbuf1, eabuf1, adsrc1, addst1, comb1,
               zbuf, bounce, acc, lsem0, lsem1, gsem):
    cid = lax.axis_index("c")
    sid = lax.axis_index("s")
    wid = sid * 2 + cid

    _zero_vmem(zbuf, 128, 16)
    for ch in range(KCH):
        q = sid + 16 * ch

        @pl.when(q < NCHUNK)
        def _():
            pltpu.sync_copy(zbuf, acc.at[pl.ds(q * 128, 128)])
    plsc.subcore_barrier()

    nb = NBAT // 32
    npairs = nb // 2
    lane = _LANE()

    def start_lin(k, sb, db, eb, sem):
        base = (wid + 32 * k) * B
        pltpu.async_copy(srch.at[pl.ds(base, B)], sb, sem)
        pltpu.async_copy(dsth.at[pl.ds(base, B)], db, sem)
        pltpu.async_copy(eah.at[pl.ds(base, B)], eb, sem)

    def wait_lin(sb, db, eb, sem):
        pltpu.make_async_copy(srch.at[pl.ds(0, B)], sb, sem).wait()
        pltpu.make_async_copy(dsth.at[pl.ds(0, B)], db, sem).wait()
        pltpu.make_async_copy(eah.at[pl.ds(0, B)], eb, sem).wait()

    def do_batch(k, sb, db, eb, asrc, adst, comb):
        base = (wid + 32 * k) * B
        g1 = pltpu.async_copy(adh.at[sb], asrc, gsem)
        g2 = pltpu.async_copy(adh.at[db], adst, gsem)
        g1.wait()
        g2.wait()

        @plsc.parallel_loop(0, B, 1, unroll=4)
        def ebody(e):
            vs = asrc[e]
            vd = adst[e]
            vds = vd.at[(lane + 4) & 15].get(mode="promise_in_bounds")
            t = vs + vds
            ex = jnp.exp(jnp.maximum(t, 0.0) + 0.2 * jnp.minimum(t, 0.0))
            ea = eb[e]
            eash = ea.at[(lane - 4) & 15].get(mode="promise_in_bounds")
            comb[e] = jnp.where(lane < 4, ex,
                                jnp.where(lane < 9, eash, 0.0))
        pltpu.sync_copy(comb, acc.at[db], add=True)
        pltpu.sync_copy(comb, coefh.at[pl.ds(base, B)])

    start_lin(0, sbuf0, dbuf0, eabuf0, lsem0)

    def pair_body(i, carry):
        k0 = 2 * i
        k1 = 2 * i + 1
        wait_lin(sbuf0, dbuf0, eabuf0, lsem0)
        start_lin(k1, sbuf1, dbuf1, eabuf1, lsem1)
        do_batch(k0, sbuf0, dbuf0, eabuf0, adsrc0, addst0, comb0)
        wait_lin(sbuf1, dbuf1, eabuf1, lsem1)

        @pl.when(i + 1 < npairs)
        def _():
            start_lin(k1 + 1, sbuf0, dbuf0, eabuf0, lsem0)

        do_batch(k1, sbuf1, dbuf1, eabuf1, adsrc1, addst1, comb1)
        return carry

    lax.fori_loop(0, npairs, pair_body, 0)

    plsc.subcore_barrier()
    for ch in range(KCH):
        q = sid + 16 * ch

        @pl.when(q < NCHUNK)
        def _():
            r0 = q * 128
            pltpu.sync_copy(acc.at[pl.ds(r0, 128)], bounce)

            @pl.when(cid == 0)
            def _():
                pltpu.sync_copy(bounce, denh.at[0].at[pl.ds(r0, 128)])

            @pl.when(cid == 1)
            def _():
                pltpu.sync_copy(bounce, denh.at[1].at[pl.ds(r0, 128)])


def _s1(srcp, dstp, eap, ad):
    kfn = pl.kernel(
        _s1_kernel, mesh=_MESH,
        compiler_params=pltpu.CompilerParams(use_tc_tiling_on_sc=False),
        out_type=[jax.ShapeDtypeStruct((E2, 16), jnp.float32),
                  jax.ShapeDtypeStruct((2, NR, 16), jnp.float32)],
        scratch_types=[
            pltpu.VMEM((B,), jnp.int32), pltpu.VMEM((B,), jnp.int32),
            pltpu.VMEM((B, 16), jnp.float32),
            pltpu.VMEM((B, 16), jnp.float32), pltpu.VMEM((B, 16), jnp.float32),
            pltpu.VMEM((B, 16), jnp.float32),
            pltpu.VMEM((B,), jnp.int32), pltpu.VMEM((B,), jnp.int32),
            pltpu.VMEM((B, 16), jnp.float32),
            pltpu.VMEM((B, 16), jnp.float32), pltpu.VMEM((B, 16), jnp.float32),
            pltpu.VMEM((B, 16), jnp.float32),
            pltpu.VMEM((128, 16), jnp.float32),
            pltpu.VMEM((128, 16), jnp.float32),
            pltpu.VMEM_SHARED((NR, 16), jnp.float32),
            pltpu.SemaphoreType.DMA, pltpu.SemaphoreType.DMA,
            pltpu.SemaphoreType.DMA,
        ])
    return kfn(srcp, dstp, eap, ad)


def _make_s2(P, C, cmap):
    def s2_kernel(srch, dsth, sch, tph, outh,
                  sbuf0, dbuf0, scbuf0, grows0,
                  sbuf1, dbuf1, scbuf1, grows1,
                  scaled, zbuf, bounce, acc, lsem0, lsem1, gsem):
        cid = lax.axis_index("c")
        sid = lax.axis_index("s")
        nb = NBAT // 16
        _zero_vmem(zbuf, 128, 32)

        def start_lin(k, sb, db, scb, sem):
            base = (sid + 16 * k) * B
            pltpu.async_copy(srch.at[pl.ds(base, B)], sb, sem)
            pltpu.async_copy(dsth.at[pl.ds(base, B)], db, sem)
            pltpu.async_copy(sch.at[pl.ds(base, B)], scb, sem)

        def wait_lin(sb, db, scb, sem):
            pltpu.make_async_copy(srch.at[pl.ds(0, B)], sb, sem).wait()
            pltpu.make_async_copy(dsth.at[pl.ds(0, B)], db, sem).wait()
            pltpu.make_async_copy(sch.at[pl.ds(0, B)], scb, sem).wait()

        for p in range(P):
            pc = 0 if p < P // 2 else 1

            @pl.when(cid == pc)
            def _process():
                for ch in range(KCH):
                    q = sid + 16 * ch

                    @pl.when(q < NCHUNK)
                    def _():
                        pltpu.sync_copy(zbuf, acc.at[pl.ds(q * 128, 128)])
                plsc.subcore_barrier()

                cidx = jnp.full((16,), cmap[p], jnp.int32)

                def start_gather(sb, grows):
                    pltpu.async_copy(tph.at[p].at[sb], grows, gsem)

                def wait_gather(sb, grows):
                    pltpu.make_async_copy(
                        tph.at[p].at[sb], grows, gsem).wait()

                def compute_batch(db, scb, grows):
                    @plsc.parallel_loop(0, B, 1, unroll=8)
                    def row_body(r):
                        sv = scb[r].at[cidx].get(mode="promise_in_bounds")
                        for j in range(2):
                            seg = grows[r, pl.ds(16 * j, 16)]
                            scaled[r, pl.ds(16 * j, 16)] = seg * sv

                    pltpu.sync_copy(scaled, acc.at[db], add=True)

                npairs = nb // 2
                start_lin(0, sbuf0, dbuf0, scbuf0, lsem0)
                wait_lin(sbuf0, dbuf0, scbuf0, lsem0)
                start_gather(sbuf0, grows0)
                start_lin(1, sbuf1, dbuf1, scbuf1, lsem1)

                def pair_body(i, carry):
                    k1 = 2 * i + 1
                    wait_lin(sbuf1, dbuf1, scbuf1, lsem1)
                    start_gather(sbuf1, grows1)
                    wait_gather(sbuf0, grows0)
                    compute_batch(dbuf0, scbuf0, grows0)

                    @pl.when(i + 1 < npairs)
                    def _():
                        start_lin(k1 + 1, sbuf0, dbuf0, scbuf0, lsem0)
                        wait_lin(sbuf0, dbuf0, scbuf0, lsem0)
                        start_gather(sbuf0, grows0)

                    wait_gather(sbuf1, grows1)
                    compute_batch(dbuf1, scbuf1, grows1)

                    @pl.when(i + 1 < npairs)
                    def _():
                        start_lin(k1 + 2, sbuf1, dbuf1, scbuf1, lsem1)

                    return carry

                lax.fori_loop(0, npairs, pair_body, 0)

                plsc.subcore_barrier()
                for ch in range(KCH):
                    q = sid + 16 * ch

                    @pl.when(q < NCHUNK)
                    def _():
                        r0 = q * 128
                        pltpu.sync_copy(acc.at[pl.ds(r0, 128)], bounce)
                        pltpu.sync_copy(bounce, outh.at[p].at[pl.ds(r0, 128)])
                plsc.subcore_barrier()

    def run(srcp, dstp, scale, tparts):
        kfn = pl.kernel(
            s2_kernel, mesh=_MESH,
            compiler_params=pltpu.CompilerParams(use_tc_tiling_on_sc=False),
            out_type=[jax.ShapeDtypeStruct((P, NR, 32), jnp.float32)],
            scratch_types=[
                pltpu.VMEM((B,), jnp.int32), pltpu.VMEM((B,), jnp.int32),
                pltpu.VMEM((B, C), jnp.float32),
                pltpu.VMEM((B, 32), jnp.float32),
                pltpu.VMEM((B,), jnp.int32), pltpu.VMEM((B,), jnp.int32),
                pltpu.VMEM((B, C), jnp.float32),
                pltpu.VMEM((B, 32), jnp.float32),
                pltpu.VMEM((B, 32), jnp.float32),
                pltpu.VMEM((128, 32), jnp.float32),
                pltpu.VMEM((128, 32), jnp.float32),
                pltpu.VMEM_SHARED((NR, 32), jnp.float32),
                pltpu.SemaphoreType.DMA, pltpu.SemaphoreType.DMA,
                pltpu.SemaphoreType.DMA,
            ])
        (out,) = kfn(srcp, dstp, scale, tparts)
        return out

    return run


_s2_gat = _make_s2(8, 16, [0, 0, 1, 1, 2, 2, 3, 3])
_s2_gcn = _make_s2(10, 16, [4, 4, 5, 5, 6, 6, 7, 7, 8, 8])


def _s3_kernel(s2h, d2h, hh, g1h, g2h, ibuf, grows, gsem):
    cid = lax.axis_index("c")
    sid = lax.axis_index("s")
    wid = sid * 2 + cid
    for k in range(ETP // (B * 32)):
        base = (wid + 32 * k) * B
        pltpu.sync_copy(s2h.at[pl.ds(base, B)], ibuf)
        pltpu.async_copy(hh.at[ibuf], grows, gsem).wait()
        pltpu.sync_copy(grows, g1h.at[pl.ds(base, B)])
        pltpu.sync_copy(d2h.at[pl.ds(base, B)], ibuf)
        pltpu.async_copy(hh.at[ibuf], grows, gsem).wait()
        pltpu.sync_copy(grows, g2h.at[pl.ds(base, B)])


def _s3(s2p, d2p, h):
    kfn = pl.kernel(
        _s3_kernel, mesh=_MESH,
        compiler_params=pltpu.CompilerParams(use_tc_tiling_on_sc=False),
        out_type=[jax.ShapeDtypeStruct((ETP, 320), jnp.float32),
                  jax.ShapeDtypeStruct((ETP, 320), jnp.float32)],
        scratch_types=[
            pltpu.VMEM((B,), jnp.int32),
            pltpu.VMEM((B, 320), jnp.float32),
            pltpu.SemaphoreType.DMA,
        ])
    return kfn(s2p, d2p, h)


def kernel(x, edge_index, edge_attr, edge_index_test, Wg, att_src, att_dst, bg,
           W_gcnA, b_gcnA, W_gcnB, b_gcnB,
           W_l0, b_l0, W_l1, b_l1, W_l2, b_l2, W_l3, b_l3, W_l4, b_l4):
    f32 = jnp.float32
    src, dst = edge_index[0], edge_index[1]
    npad = E2 - E
    srcp = jnp.concatenate([src, jnp.zeros((npad,), src.dtype)])
    dstp = jnp.concatenate([dst, jnp.full((npad,), N, dst.dtype)])
    eap = jnp.concatenate([edge_attr, jnp.zeros((npad, NEA), f32)], axis=0)
    eap = jnp.pad(eap, ((0, 0), (0, 16 - NEA)))

    xp = jnp.pad(x, ((0, 0), (0, 128 - DIN)))
    wgp = jnp.pad(Wg, ((0, 128 - DIN), (0, 0)))
    atts = att_src.reshape(1, H * HID)
    attd = att_dst.reshape(1, H * HID)

    hp_parts, ad = _m1(xp, wgp, atts, attd)
    adp = jnp.concatenate([ad, jnp.zeros((NPAD, 16), f32)], axis=0)

    coef, den2 = _s1(srcp, dstp, eap, adp)

    num_parts = _s2_gat(srcp, dstp, coef, hp_parts)

    wcat_a = jnp.transpose(W_gcnA, (1, 0, 2)).reshape(H * HID, NEA * HID)
    hp, dv = _gatepi(num_parts, hp_parts, ad, den2, bg.reshape(1, -1), wcat_a)

    for l in range(4):
        s_parts = _s2_gcn(srcp, dstp, coef, hp)
        bl = (b_gcnA if l == 0 else b_gcnB[l - 1]).reshape(1, -1)
        wcat = jnp.transpose(W_gcnB[l], (1, 0, 2)).reshape(NEA * HID, NEA * HID)
        hp = _layer(s_parts, hp, dv, bl, wcat)

    s_parts = _s2_gcn(srcp, dstp, coef, hp)
    h_final = _final(s_parts, hp, dv, b_gcnB[3].reshape(1, -1))

    s2i, d2i = edge_index_test[0], edge_index_test[1]
    tpad = ETP - ET
    s2p = jnp.concatenate([s2i, jnp.zeros((tpad,), s2i.dtype)])
    d2p = jnp.concatenate([d2i, jnp.zeros((tpad,), d2i.dtype)])
    g1, g2 = _s3(s2p, d2p, h_final)

    out = _mlp(g1, g2,
               W_l0, b_l0.reshape(1, -1), W_l1, b_l1.reshape(1, -1),
               W_l2, b_l2.reshape(1, -1), W_l3, b_l3.reshape(1, -1),
               W_l4, b_l4.reshape(1, -1))
    return out[:ET]
